# baseline (speedup 1.0000x reference)
import functools

import jax
import jax.numpy as jnp
from jax import lax
from jax.experimental import pallas as pl
from jax.experimental.pallas import tpu as pltpu
from jax.experimental.pallas import tpu_sc as plsc

B, S, D, K = 4, 4096, 1024, 512
L = 16
CH = S // 8
NCV = CH // L
SLAB = CH + L
NSV = SLAB // L
ROWS_PER_TILE = K // 8
SPAD = K + 2 * L


def _score_body(t_ref, w_ref, s_ref, meta_ref):
    t2 = t_ref[...].reshape(S, D)
    s = lax.dot_general(w_ref[...], t2, (((1,), (1,)), ((), ())),
                        preferred_element_type=jnp.float32)
    s_ref[...] = s.reshape(1, 1, S)
    bi = lax.bitcast_convert_type(s, jnp.int32)
    key = bi ^ ((bi >> 31) & jnp.int32(0x7FFFFFFF))
    u = lax.bitcast_convert_type(key, jnp.uint32) ^ jnp.uint32(0x80000000)
    t = jnp.uint32(0)
    for step in range(32):
        trial = t | jnp.uint32(1 << (31 - step))
        cnt = jnp.sum((u >= trial).astype(jnp.int32))
        t = jnp.where(cnt >= K, trial, t)
    count_gt = jnp.sum((u > t).astype(jnp.int32))
    m = jnp.int32(K) - count_gt
    t_i = lax.bitcast_convert_type(t, jnp.int32)
    io = lax.broadcasted_iota(jnp.int32, (1, 1, 128), 2)
    meta_ref[...] = jnp.where(
        io == 0, t_i, jnp.where(io == 1, count_gt,
                                jnp.where(io == 2, m, jnp.int32(0))))


def _scores_tc(token, w_row):
    return pl.pallas_call(
        _score_body,
        grid=(B,),
        in_specs=[
            pl.BlockSpec((1, S, D), lambda b: (b, 0, 0)),
            pl.BlockSpec((1, D), lambda b: (0, 0)),
        ],
        out_specs=[
            pl.BlockSpec((1, 1, S), lambda b: (b, 0, 0)),
            pl.BlockSpec((1, 1, 128), lambda b: (b, 0, 0)),
        ],
        out_shape=[
            jax.ShapeDtypeStruct((B, 1, S), jnp.float32),
            jax.ShapeDtypeStruct((B, 1, 128), jnp.int32),
        ],
    )(token, w_row)


_mesh = plsc.VectorSubcoreMesh(core_axis_name="c", subcore_axis_name="s")


@functools.partial(
    pl.kernel,
    mesh=_mesh,
    compiler_params=pltpu.CompilerParams(needs_layout_passes=False),
    out_type=jax.ShapeDtypeStruct((B * K, D), jnp.float32),
    scratch_types=[
        pltpu.VMEM((CH,), jnp.float32),
        pltpu.VMEM((128,), jnp.int32),
        pltpu.VMEM((SLAB,), jnp.uint32),
        pltpu.VMEM((SLAB,), jnp.int32),
        pltpu.VMEM((SLAB,), jnp.int32),
        pltpu.VMEM((16,), jnp.int32),
        pltpu.VMEM((16 * L,), jnp.int32),
        pltpu.VMEM((8 * SLAB,), jnp.uint32),
        pltpu.VMEM((8 * SLAB,), jnp.int32),
        pltpu.VMEM((SLAB,), jnp.int32),
        pltpu.VMEM((SLAB,), jnp.int32),
        pltpu.VMEM((SLAB,), jnp.int32),
        pltpu.VMEM((SLAB,), jnp.int32),
        pltpu.VMEM((ROWS_PER_TILE,), jnp.int32),
        pltpu.VMEM((ROWS_PER_TILE, D), jnp.float32),
        pltpu.VMEM_SHARED((2 * 8 * SLAB,), jnp.uint32),
        pltpu.VMEM_SHARED((2 * 8 * SLAB,), jnp.int32),
        pltpu.VMEM_SHARED((2 * 8 * SLAB,), jnp.int32),
        pltpu.VMEM_SHARED((16 * L,), jnp.int32),
        pltpu.VMEM_SHARED((2 * SPAD,), jnp.int32),
        pltpu.SemaphoreType.DMA,
    ],
)
def _sc_topk_gather(scores_hbm, meta_hbm, token_hbm, out_hbm,
                    scf_v, meta128_v, lcu_v, lcidx_v, ltied_v, cnt_v,
                    allcnt_v, acu_v, acidx_v, rank_v, rowid_v,
                    trank_v, trowid_v, idx_v, rows_v,
                    sh_cu, sh_cidx, sh_tied, sh_cnt, sh_sorted, sem):
    cid = lax.axis_index("c")
    sid = lax.axis_index("s")
    iota = lax.iota(jnp.int32, L)
    b1 = sid // 8
    chunk = sid % 8
    b = 2 * cid + b1
    slab0 = (b1 * 8 + chunk) * SLAB

    pltpu.sync_copy(scores_hbm.at[b].at[pl.ds(chunk * CH, CH)], scf_v)
    pltpu.sync_copy(meta_hbm.at[b], meta128_v)
    mv = meta128_v[pl.ds(0, L)]
    t = lax.bitcast_convert_type(mv, jnp.uint32)[0]
    count_gt = mv[1]
    m = mv[2]

    def zfill(i, carry):
        lcu_v[pl.ds(i * L, L)] = jnp.zeros((L,), jnp.uint32)
        return carry
    lax.fori_loop(0, NSV, zfill, 0)

    def compact_body(i, carry):
        og, oe = carry
        f = scf_v[pl.ds(i * L, L)]
        bi = lax.bitcast_convert_type(f, jnp.int32)
        keyi = bi ^ ((bi >> 31) & jnp.int32(0x7FFFFFFF))
        x = lax.bitcast_convert_type(keyi, jnp.uint32) ^ jnp.uint32(0x80000000)
        idxv = chunk * CH + i * L + iota
        gt = x > t
        eq = x == t
        plsc.store_compressed(lcu_v.at[pl.ds(og, L)], x, mask=gt)
        plsc.store_compressed(lcidx_v.at[pl.ds(og, L)], idxv, mask=gt)
        plsc.store_compressed(ltied_v.at[pl.ds(oe, L)], idxv, mask=eq)
        return (og + plsc.all_reduce_population_count(gt)[0],
                oe + plsc.all_reduce_population_count(eq)[0])
    cg, ce = lax.fori_loop(0, NCV, compact_body,
                           (jnp.int32(0), jnp.int32(0)))

    cnt_v[pl.ds(0, L)] = jnp.where(
        iota == 0, cg, jnp.where(iota == 1, ce, jnp.int32(0)))
    pltpu.sync_copy(lcu_v, sh_cu.at[pl.ds(slab0, SLAB)])
    pltpu.sync_copy(lcidx_v, sh_cidx.at[pl.ds(slab0, SLAB)])
    pltpu.sync_copy(ltied_v, sh_tied.at[pl.ds(slab0, SLAB)])
    pltpu.sync_copy(cnt_v, sh_cnt.at[pl.ds(sid * L, L)])

    plsc.subcore_barrier()

    base = b * jnp.int32(S)
    slab = b1 * jnp.int32(SPAD)

    pltpu.sync_copy(sh_cu.at[pl.ds(b1 * 8 * SLAB, 8 * SLAB)], acu_v)
    pltpu.sync_copy(sh_cidx.at[pl.ds(b1 * 8 * SLAB, 8 * SLAB)], acidx_v)
    pltpu.sync_copy(sh_cnt, allcnt_v)

    cgs = []
    pe = jnp.int32(0)
    for c in range(8):
        cv = allcnt_v[pl.ds((b1 * 8 + c) * L, L)]
        cgs.append(cv[0])
        pe = pe + jnp.where(jnp.int32(c) < chunk, cv[1], jnp.int32(0))

    def rfill(i, carry):
        rank_v[pl.ds(i * L, L)] = slab + jnp.int32(K) + iota
        trank_v[pl.ds(i * L, L)] = slab + jnp.int32(K) + iota
        return carry
    lax.fori_loop(0, NSV, rfill, 0)

    nloc = (cg + (L - 1)) // L

    def rank_chunk(gi, carry):
        iv = lcu_v[pl.ds(gi * L, L)]
        iidx = lcidx_v[pl.ds(gi * L, L)]
        r = jnp.zeros((L,), jnp.int32)
        for sl in range(8):
            ns = (cgs[sl] + (L - 1)) // L

            def rank_inner(jv, r, sl=sl, iv=iv, iidx=iidx):
                uj16 = acu_v[pl.ds(sl * SLAB + jv * L, L)]
                ij16 = acidx_v[pl.ds(sl * SLAB + jv * L, L)]
                for lane in range(L):
                    uj = uj16[lane]
                    ij = ij16[lane]
                    hit = (uj > iv) | ((uj == iv) & (ij < iidx))
                    r = r + hit.astype(jnp.int32)
                return r
            r = lax.fori_loop(0, ns, rank_inner, r)
        lane_ok = (gi * L + iota) < cg
        rank_v[pl.ds(gi * L, L)] = slab + jnp.where(
            lane_ok, r, jnp.int32(K) + iota)
        rowid_v[pl.ds(gi * L, L)] = iidx + base
        return carry
    lax.fori_loop(0, nloc, rank_chunk, 0)
    pltpu.sync_copy(rowid_v, sh_sorted.at[rank_v])

    ntv = (ce + (L - 1)) // L

    def tied_chunk(tv, carry):
        p = pe + tv * L + iota
        ti = ltied_v[pl.ds(tv * L, L)]
        lane_ok = ((tv * L + iota) < ce) & (p < m)
        trank_v[pl.ds(tv * L, L)] = slab + jnp.where(
            lane_ok, count_gt + p, jnp.int32(K) + iota)
        trowid_v[pl.ds(tv * L, L)] = ti + base
        return carry
    lax.fori_loop(0, ntv, tied_chunk, 0)
    pltpu.sync_copy(trowid_v, sh_sorted.at[trank_v])

    plsc.subcore_barrier()

    pltpu.sync_copy(
        sh_sorted.at[pl.ds(slab + chunk * ROWS_PER_TILE, ROWS_PER_TILE)],
        idx_v)
    pltpu.async_copy(token_hbm.at[idx_v], rows_v, sem).wait()
    row0 = b * K + chunk * ROWS_PER_TILE
    pltpu.sync_copy(rows_v, out_hbm.at[pl.ds(row0, ROWS_PER_TILE)])


def kernel(token, mask, label, w_att):
    scores3, meta3 = _scores_tc(token, w_att.reshape(1, D))
    token2 = token.reshape(B * S, D)
    out2 = _sc_topk_gather(scores3.reshape(B, S), meta3.reshape(B, 128),
                           token2)
    return out2.reshape(B, K, D)
